# baseline (speedup 1.0000x reference)
import functools

import jax
import jax.numpy as jnp
from jax import lax
from jax.experimental import pallas as pl
from jax.experimental.pallas import tpu as pltpu
from jax.experimental.pallas import tpu_sc as plsc

B = 16384
D = 128
NC = 2
NS = 16
NW = NC * NS
BPW = B // NW
CH = 64
NCH = BPW // CH

_mesh = plsc.VectorSubcoreMesh(core_axis_name="c", subcore_axis_name="s")


@functools.partial(
    pl.kernel,
    mesh=_mesh,
    compiler_params=pltpu.CompilerParams(needs_layout_passes=False),
    out_type=jax.ShapeDtypeStruct((B,), jnp.float32),
    scratch_types=[
        pltpu.VMEM((BPW,), jnp.int32),
        pltpu.VMEM((BPW,), jnp.int32),
        pltpu.VMEM((CH, D), jnp.float32),
        pltpu.VMEM((CH, D), jnp.float32),
        pltpu.VMEM((CH, D), jnp.float32),
        pltpu.VMEM((CH, D), jnp.float32),
        pltpu.VMEM((CH, D), jnp.float32),
        pltpu.VMEM((CH, D), jnp.float32),
        pltpu.VMEM((CH, D), jnp.float32),
        pltpu.VMEM((CH, D), jnp.float32),
        pltpu.VMEM((CH, D), jnp.float32),
        pltpu.VMEM((CH, D), jnp.float32),
        pltpu.VMEM((CH, D), jnp.float32),
        pltpu.VMEM((CH, D), jnp.float32),
        pltpu.VMEM((BPW,), jnp.float32),
        pltpu.SemaphoreType.DMA,
        pltpu.SemaphoreType.DMA,
        pltpu.SemaphoreType.DMA,
        pltpu.SemaphoreType.DMA,
        pltpu.SemaphoreType.DMA,
        pltpu.SemaphoreType.DMA,
        pltpu.SemaphoreType.DMA,
        pltpu.SemaphoreType.DMA,
        pltpu.SemaphoreType.DMA,
        pltpu.SemaphoreType.DMA,
        pltpu.SemaphoreType.DMA,
        pltpu.SemaphoreType.DMA,
        pltpu.SemaphoreType.DMA,
        pltpu.SemaphoreType.DMA,
    ],
)
def _sc_dot_kernel(user_hbm, item_hbm, uf_hbm, if_hbm, out_hbm,
                   uidx_v, iidx_v, u0_v, u1_v, u2_v, u3_v, u4_v, u5_v,
                   v0_v, v1_v, v2_v, v3_v, v4_v, v5_v, o_v,
                   sem_u0, sem_u1, sem_u2, sem_u3, sem_u4, sem_u5,
                   sem_v0, sem_v1, sem_v2, sem_v3, sem_v4, sem_v5,
                   sem_i, sem_o):
    wid = lax.axis_index("s") * NC + lax.axis_index("c")
    base = wid * BPW
    u_bufs = (u0_v, u1_v, u2_v, u3_v, u4_v, u5_v)
    v_bufs = (v0_v, v1_v, v2_v, v3_v, v4_v, v5_v)
    u_sems = (sem_u0, sem_u1, sem_u2, sem_u3, sem_u4, sem_u5)
    v_sems = (sem_v0, sem_v1, sem_v2, sem_v3, sem_v4, sem_v5)
    lane = lax.iota(jnp.int32, 16)

    cpi_u = pltpu.async_copy(user_hbm.at[pl.ds(base, BPW)], uidx_v, sem_i)
    cpi_i = pltpu.async_copy(item_hbm.at[pl.ds(base, BPW)], iidx_v, sem_i)
    cpi_u.wait()
    cpi_i.wait()

    def issue(c, p):
        pltpu.async_copy(uf_hbm.at[uidx_v.at[pl.ds(c * CH, CH)]],
                         u_bufs[p], u_sems[p])
        pltpu.async_copy(if_hbm.at[iidx_v.at[pl.ds(c * CH, CH)]],
                         v_bufs[p], v_sems[p])

    def drain(c, p):
        pltpu.make_async_copy(uf_hbm.at[uidx_v.at[pl.ds(c * CH, CH)]],
                              u_bufs[p], u_sems[p]).wait()
        pltpu.make_async_copy(if_hbm.at[iidx_v.at[pl.ds(c * CH, CH)]],
                              v_bufs[p], v_sems[p]).wait()

    def compute(c, p):
        u_v, v_v = u_bufs[p], v_bufs[p]

        def group_body(g, carry):
            rows = g * 16 + lane

            zero = jnp.zeros((16,), jnp.float32)

            @plsc.parallel_loop(0, D, step=2, unroll=4,
                                carry=(zero, zero, lane))
            def acc_loop(d, state, rows=rows):
                a0, a1, col = state
                c1 = (col + 1) & (D - 1)
                a0 = a0 + plsc.load_gather(u_v, [rows, col]) * \
                    plsc.load_gather(v_v, [rows, col])
                a1 = a1 + plsc.load_gather(u_v, [rows, c1]) * \
                    plsc.load_gather(v_v, [rows, c1])
                return a0, a1, (col + 2) & (D - 1)

            o_v[pl.ds(c * CH + g * 16, 16)] = acc_loop[0] + acc_loop[1]
            return carry

        lax.fori_loop(0, CH // 16, group_body, 0)
        pltpu.async_copy(o_v.at[pl.ds(c * CH, CH)],
                         out_hbm.at[pl.ds(base + c * CH, CH)], sem_o)

    for c in range(5):
        issue(c, c)
    for c in range(NCH):
        drain(c, c % 6)
        if c + 5 < NCH:
            issue(c + 5, (c + 5) % 6)
        compute(c, c % 6)

    def drain_out(c, carry):
        pltpu.make_async_copy(o_v.at[pl.ds(c * CH, CH)],
                              out_hbm.at[pl.ds(base + c * CH, CH)],
                              sem_o).wait()
        return carry

    lax.fori_loop(0, NCH, drain_out, 0)


def kernel(user, item, user_factors, item_factors):
    return _sc_dot_kernel(user.astype(jnp.int32), item.astype(jnp.int32),
                          user_factors, item_factors)


# speedup vs baseline: 1.0162x; 1.0162x over previous
import functools

import jax
import jax.numpy as jnp
from jax import lax
from jax.experimental import pallas as pl
from jax.experimental.pallas import tpu as pltpu
from jax.experimental.pallas import tpu_sc as plsc

B = 16384
D = 128
NC = 2
NS = 16
NW = NC * NS
BPW = B // NW
CH = 64
NCH = BPW // CH

_mesh = plsc.VectorSubcoreMesh(core_axis_name="c", subcore_axis_name="s")


@functools.partial(
    pl.kernel,
    mesh=_mesh,
    compiler_params=pltpu.CompilerParams(needs_layout_passes=False),
    out_type=jax.ShapeDtypeStruct((B,), jnp.float32),
    scratch_types=[
        pltpu.VMEM((BPW,), jnp.int32),
        pltpu.VMEM((BPW,), jnp.int32),
        pltpu.VMEM((CH, D), jnp.float32),
        pltpu.VMEM((CH, D), jnp.float32),
        pltpu.VMEM((CH, D), jnp.float32),
        pltpu.VMEM((CH, D), jnp.float32),
        pltpu.VMEM((CH, D), jnp.float32),
        pltpu.VMEM((CH, D), jnp.float32),
        pltpu.VMEM((CH, D), jnp.float32),
        pltpu.VMEM((CH, D), jnp.float32),
        pltpu.VMEM((BPW,), jnp.float32),
        pltpu.SemaphoreType.DMA,
        pltpu.SemaphoreType.DMA,
        pltpu.SemaphoreType.DMA,
        pltpu.SemaphoreType.DMA,
        pltpu.SemaphoreType.DMA,
        pltpu.SemaphoreType.DMA,
        pltpu.SemaphoreType.DMA,
        pltpu.SemaphoreType.DMA,
        pltpu.SemaphoreType.DMA,
        pltpu.SemaphoreType.DMA,
    ],
)
def _sc_dot_kernel(user_hbm, item_hbm, uf_hbm, if_hbm, out_hbm,
                   uidx_v, iidx_v, u0_v, u1_v, u2_v, u3_v,
                   v0_v, v1_v, v2_v, v3_v, o_v,
                   sem_u0, sem_u1, sem_u2, sem_u3,
                   sem_v0, sem_v1, sem_v2, sem_v3,
                   sem_i, sem_o):
    wid = lax.axis_index("s") * NC + lax.axis_index("c")
    base = wid * BPW
    u_bufs = (u0_v, u1_v, u2_v, u3_v)
    v_bufs = (v0_v, v1_v, v2_v, v3_v)
    u_sems = (sem_u0, sem_u1, sem_u2, sem_u3)
    v_sems = (sem_v0, sem_v1, sem_v2, sem_v3)
    lane = lax.iota(jnp.int32, 16)

    cpi_u = pltpu.async_copy(user_hbm.at[pl.ds(base, BPW)], uidx_v, sem_i)
    cpi_i = pltpu.async_copy(item_hbm.at[pl.ds(base, BPW)], iidx_v, sem_i)
    cpi_u.wait()
    cpi_i.wait()

    def issue(c, p):
        pltpu.async_copy(uf_hbm.at[uidx_v.at[pl.ds(c * CH, CH)]],
                         u_bufs[p], u_sems[p])
        pltpu.async_copy(if_hbm.at[iidx_v.at[pl.ds(c * CH, CH)]],
                         v_bufs[p], v_sems[p])

    def drain(c, p):
        pltpu.make_async_copy(uf_hbm.at[uidx_v.at[pl.ds(c * CH, CH)]],
                              u_bufs[p], u_sems[p]).wait()
        pltpu.make_async_copy(if_hbm.at[iidx_v.at[pl.ds(c * CH, CH)]],
                              v_bufs[p], v_sems[p]).wait()

    def compute(c, p):
        u_v, v_v = u_bufs[p], v_bufs[p]

        def group_body(g, carry):
            rows = g * 16 + lane

            zero = jnp.zeros((16,), jnp.float32)

            @plsc.parallel_loop(0, D, step=2, unroll=4,
                                carry=(zero, zero, lane))
            def acc_loop(d, state, rows=rows):
                a0, a1, col = state
                c1 = (col + 1) & (D - 1)
                a0 = a0 + plsc.load_gather(u_v, [rows, col]) * \
                    plsc.load_gather(v_v, [rows, col])
                a1 = a1 + plsc.load_gather(u_v, [rows, c1]) * \
                    plsc.load_gather(v_v, [rows, c1])
                return a0, a1, (col + 2) & (D - 1)

            o_v[pl.ds(c * CH + g * 16, 16)] = acc_loop[0] + acc_loop[1]
            return carry

        lax.fori_loop(0, CH // 16, group_body, 0)
        pltpu.async_copy(o_v.at[pl.ds(c * CH, CH)],
                         out_hbm.at[pl.ds(base + c * CH, CH)], sem_o)

    issue(0, 0)
    issue(1, 1)
    issue(2, 2)
    for c in range(NCH):
        drain(c, c % 4)
        if c + 3 < NCH:
            issue(c + 3, (c + 3) % 4)
        compute(c, c % 4)

    def drain_out(c, carry):
        pltpu.make_async_copy(o_v.at[pl.ds(c * CH, CH)],
                              out_hbm.at[pl.ds(base + c * CH, CH)],
                              sem_o).wait()
        return carry

    lax.fori_loop(0, NCH, drain_out, 0)


def kernel(user, item, user_factors, item_factors):
    return _sc_dot_kernel(user.astype(jnp.int32), item.astype(jnp.int32),
                          user_factors, item_factors)
